# baseline (speedup 1.0000x reference)
import jax
import jax.numpy as jnp
from jax import lax
from jax.experimental import pallas as pl
from jax.experimental.pallas import tpu as pltpu
from jax.experimental.pallas import tpu_sc as plsc

BATCH = 16384
EMB = 64
H1 = 128
H2 = 64

NC = 2
NS = 16
NW = NC * NS
B_PER_W = BATCH // NW
CHUNK = 128
NCHUNK = B_PER_W // CHUNK

Q = 8192
SLAB = 4 * Q
NSTEP = -(-1000000 // SLAB)
PACK_ROWS = NSTEP * Q

SC_SLABS = 9
TC_SLABS = NSTEP - SC_SLABS
SC_ROWS = SC_SLABS * Q
TC_ROWS = TC_SLABS * Q
RPW = Q // NW


def _b16hi(x):
    u = lax.bitcast_convert_type(x, jnp.uint32)
    return (u + jnp.uint32(0x8000)) & jnp.uint32(0xFFFF0000)


def _pack_body(x_ref, out_ref):
    a = _b16hi(x_ref[:, :Q].T)
    b = _b16hi(x_ref[:, Q:2 * Q].T)
    c = _b16hi(x_ref[:, 2 * Q:3 * Q].T)
    d = _b16hi(x_ref[:, 3 * Q:].T)
    left = a | lax.shift_right_logical(b, jnp.uint32(16))
    right = c | lax.shift_right_logical(d, jnp.uint32(16))
    out_ref[:, :EMB] = lax.bitcast_convert_type(left, jnp.float32)
    out_ref[:, EMB:] = lax.bitcast_convert_type(right, jnp.float32)


def _tc_pack(tt):
    return pl.pallas_call(
        _pack_body,
        grid=(TC_SLABS,),
        in_specs=[pl.BlockSpec((EMB, SLAB), lambda i: (0, i + SC_SLABS))],
        out_specs=pl.BlockSpec((Q, 2 * EMB), lambda i: (i, 0)),
        out_shape=jax.ShapeDtypeStruct((TC_ROWS, 2 * EMB), jnp.float32),
        compiler_params=pltpu.CompilerParams(
            dimension_semantics=("parallel",)),
    )(tt)


def _pack16(a, b):
    au = lax.bitcast_convert_type(a, jnp.uint32)
    bu = lax.bitcast_convert_type(b, jnp.uint32)
    hi = (au + jnp.uint32(0x8000)) & jnp.uint32(0xFFFF0000)
    lo = lax.shift_right_logical(bu + jnp.uint32(0x8000), jnp.uint32(16))
    return lax.bitcast_convert_type(hi | lo, jnp.float32)


def _sc_pack_body(ut, it, pu_hbm, pi_hbm, cha, chb, chc, chd, outb):
    wid = lax.axis_index("s") * NC + lax.axis_index("c")
    cvecs = [lax.iota(jnp.int32, 16) + 16 * k for k in range(4)]
    for src, dst in ((ut, pu_hbm), (it, pi_hbm)):
        @pl.loop(0, SC_SLABS)
        def _(s, src=src, dst=dst):
            base = s * SLAB + wid * RPW
            pltpu.sync_copy(src.at[:, pl.ds(base, RPW)], cha)
            pltpu.sync_copy(src.at[:, pl.ds(base + Q, RPW)], chb)
            pltpu.sync_copy(src.at[:, pl.ds(base + 2 * Q, RPW)], chc)
            pltpu.sync_copy(src.at[:, pl.ds(base + 3 * Q, RPW)], chd)

            @pl.loop(0, RPW)
            def _(mm):
                mv = jnp.full((16,), 0, jnp.int32) + mm
                for k in range(4):
                    a = plsc.load_gather(cha, [cvecs[k], mv])
                    b = plsc.load_gather(chb, [cvecs[k], mv])
                    c = plsc.load_gather(chc, [cvecs[k], mv])
                    d = plsc.load_gather(chd, [cvecs[k], mv])
                    outb[mm, pl.ds(16 * k, 16)] = _pack16(a, b)
                    outb[mm, pl.ds(EMB + 16 * k, 16)] = _pack16(c, d)

            pltpu.sync_copy(outb, dst.at[pl.ds(s * Q + wid * RPW, RPW)])


def _sc_pack(ut, it):
    mesh = plsc.VectorSubcoreMesh(core_axis_name="c", subcore_axis_name="s")
    out_t = (jax.ShapeDtypeStruct((SC_ROWS, 2 * EMB), jnp.float32),
             jax.ShapeDtypeStruct((SC_ROWS, 2 * EMB), jnp.float32))
    scratch = [
        pltpu.VMEM((EMB, RPW), jnp.float32),
        pltpu.VMEM((EMB, RPW), jnp.float32),
        pltpu.VMEM((EMB, RPW), jnp.float32),
        pltpu.VMEM((EMB, RPW), jnp.float32),
        pltpu.VMEM((RPW, 2 * EMB), jnp.float32),
    ]
    cp = pltpu.CompilerParams(use_tc_tiling_on_sc=True)
    if "needs_layout_passes" in pltpu.CompilerParams.__dataclass_fields__:
        import dataclasses
        cp = dataclasses.replace(cp, needs_layout_passes=False)
    k = pl.kernel(_sc_pack_body, out_type=out_t, mesh=mesh,
                  scratch_types=scratch, compiler_params=cp)
    return k(ut, it)


def _gather_body(ptc_hbm, psc_hbm, jtc_hbm, jsc_hbm, xtc_hbm, xsc_hbm,
                 jtc_v, jsc_v, rows_v, sem):
    wid = lax.axis_index("s") * NC + lax.axis_index("c")
    base = wid * B_PER_W
    pltpu.sync_copy(jtc_hbm.at[wid], jtc_v)
    pltpu.sync_copy(jsc_hbm.at[wid], jsc_v)
    copies = []
    for m in range(NCHUNK):
        copies.append(pltpu.async_copy(
            ptc_hbm.at[jtc_v.at[m]], rows_v.at[pl.ds(m * CHUNK, CHUNK)], sem))
    for c in copies:
        c.wait()
    pltpu.sync_copy(rows_v, xtc_hbm.at[pl.ds(base, B_PER_W)])
    copies = []
    for m in range(NCHUNK):
        copies.append(pltpu.async_copy(
            psc_hbm.at[jsc_v.at[m]], rows_v.at[pl.ds(m * CHUNK, CHUNK)], sem))
    for c in copies:
        c.wait()
    pltpu.sync_copy(rows_v, xsc_hbm.at[pl.ds(base, B_PER_W)])


def _sc_gather(ptc, psc, jtc3, jsc3):
    mesh = plsc.VectorSubcoreMesh(core_axis_name="c", subcore_axis_name="s")
    out_t = (jax.ShapeDtypeStruct((BATCH, 2 * EMB), jnp.float32),
             jax.ShapeDtypeStruct((BATCH, 2 * EMB), jnp.float32))
    scratch = [
        pltpu.VMEM((NCHUNK, CHUNK), jnp.int32),
        pltpu.VMEM((NCHUNK, CHUNK), jnp.int32),
        pltpu.VMEM((B_PER_W, 2 * EMB), jnp.float32),
        pltpu.SemaphoreType.DMA,
    ]
    k = pl.kernel(_gather_body, out_type=out_t, mesh=mesh,
                  scratch_types=scratch,
                  compiler_params=pltpu.CompilerParams(
                      use_tc_tiling_on_sc=True))
    return k(ptc, psc, jtc3, jsc3)


BLK = 2048


def _unpack(xtc_ref, xsc_ref, sc_ref, half_ref, lo_ref):
    xt = lax.bitcast_convert_type(xtc_ref[...], jnp.uint32)
    xs = lax.bitcast_convert_type(xsc_ref[...], jnp.uint32)
    sc = sc_ref[...] > 0.5
    half = half_ref[...] > 0.5
    lo = lo_ref[...] > 0.5
    xi = jnp.where(sc, xs, xt)
    w = jnp.where(half, xi[:, EMB:], xi[:, :EMB])
    bits = jnp.where(lo, lax.shift_left(w, jnp.uint32(16)),
                     w & jnp.uint32(0xFFFF0000))
    return lax.bitcast_convert_type(bits, jnp.float32)


def _mlp_body(xut_ref, xus_ref, xit_ref, xis_ref, us_ref, uh_ref, ul_ref,
              is_ref, ih_ref, il_ref, w1u_ref,
              w1i_ref, b1_ref, w2_ref, b2_ref, w3_ref, b3_ref, out_ref):
    ue = _unpack(xut_ref, xus_ref, us_ref, uh_ref, ul_ref)
    ie = _unpack(xit_ref, xis_ref, is_ref, ih_ref, il_ref)
    h1 = jnp.dot(ue, w1u_ref[...], preferred_element_type=jnp.float32)
    h1 += jnp.dot(ie, w1i_ref[...], preferred_element_type=jnp.float32)
    h1 = jnp.maximum(h1 + b1_ref[...], 0.0)
    h2 = jnp.dot(h1, w2_ref[...], preferred_element_type=jnp.float32)
    h2 = jnp.maximum(h2 + b2_ref[...], 0.0)
    out = jnp.dot(h2, w3_ref[...], preferred_element_type=jnp.float32)
    out_ref[...] = out[:, 0] + b3_ref[0]


def _tc_mlp(xut, xus, xit, xis, us, uh, ul, is_, ih, il,
            W1u, W1i, b1r, W2, b2r, W3, b3):
    grid = (BATCH // BLK,)
    const = lambda shape: pl.BlockSpec(shape, lambda i: tuple(0 for _ in shape))
    col = pl.BlockSpec((BLK, 1), lambda i: (i, 0))
    xspec = pl.BlockSpec((BLK, 2 * EMB), lambda i: (i, 0))
    return pl.pallas_call(
        _mlp_body,
        grid=grid,
        in_specs=[
            xspec, xspec, xspec, xspec,
            col, col, col, col, col, col,
            const((EMB, H1)),
            const((EMB, H1)),
            const((1, H1)),
            const((H1, H2)),
            const((1, H2)),
            const((H2, 1)),
            const((1,)),
        ],
        out_specs=pl.BlockSpec((BLK,), lambda i: (i,)),
        out_shape=jax.ShapeDtypeStruct((BATCH,), jnp.float32),
    )(xut, xus, xit, xis, us, uh, ul, is_, ih, il,
      W1u, W1i, b1r, W2, b2r, W3, b3)


QB = Q.bit_length() - 1


def _row_ids(r):
    return ((r >> (QB + 2)) << QB) | (r & (Q - 1))


def _colmask(bits):
    return bits.astype(jnp.float32).reshape(BATCH, 1)


@jax.jit
def kernel(user_table, item_table, W1, b1, W2, b2, W3, b3, user_input,
           item_input):
    ujdx = _row_ids(user_input)
    ijdx = _row_ids(item_input)
    ufrom_sc = ujdx < SC_ROWS
    ifrom_sc = ijdx < SC_ROWS
    ujtc = jnp.where(ufrom_sc, 0, ujdx - SC_ROWS)
    ujsc = jnp.where(ufrom_sc, ujdx, 0)
    ijtc = jnp.where(ifrom_sc, 0, ijdx - SC_ROWS)
    ijsc = jnp.where(ifrom_sc, ijdx, 0)
    shp = (NW, NCHUNK, CHUNK)
    psu, psi = _sc_pack(user_table.T, item_table.T)
    ptu = _tc_pack(user_table.T)
    xut, xus = _sc_gather(ptu, psu, ujtc.reshape(shp), ujsc.reshape(shp))
    pti = _tc_pack(item_table.T)
    xit, xis = _sc_gather(pti, psi, ijtc.reshape(shp), ijsc.reshape(shp))
    us = _colmask(ufrom_sc)
    is_ = _colmask(ifrom_sc)
    uh = _colmask((user_input >> (QB + 1)) & 1)
    ul = _colmask((user_input >> QB) & 1)
    ih = _colmask((item_input >> (QB + 1)) & 1)
    il = _colmask((item_input >> QB) & 1)
    return _tc_mlp(xut, xus, xit, xis, us, uh, ul, is_, ih, il,
                   W1[:EMB], W1[EMB:],
                   b1.reshape(1, H1), W2, b2.reshape(1, H2), W3, b3)


# speedup vs baseline: 5.4565x; 5.4565x over previous
import jax
import jax.numpy as jnp
from jax import lax
from jax.experimental import pallas as pl
from jax.experimental.pallas import tpu as pltpu
from jax.experimental.pallas import tpu_sc as plsc

BATCH = 16384
EMB = 64
H1 = 128
H2 = 64

NC = 2
NS = 16
NW = NC * NS
B_PER_W = BATCH // NW
CHUNK = 128
NCHUNK = B_PER_W // CHUNK

Q = 8192
SLAB = 4 * Q
NSTEP = -(-1000000 // SLAB)
PACK_ROWS = NSTEP * Q


def _b16hi(x):
    u = lax.bitcast_convert_type(x, jnp.uint32)
    return (u + jnp.uint32(0x8000)) & jnp.uint32(0xFFFF0000)


def _pack_body(x_ref, out_ref):
    a = _b16hi(x_ref[:, :Q].T)
    b = _b16hi(x_ref[:, Q:2 * Q].T)
    c = _b16hi(x_ref[:, 2 * Q:3 * Q].T)
    d = _b16hi(x_ref[:, 3 * Q:].T)
    left = a | lax.shift_right_logical(b, jnp.uint32(16))
    right = c | lax.shift_right_logical(d, jnp.uint32(16))
    out_ref[:, :EMB] = lax.bitcast_convert_type(left, jnp.float32)
    out_ref[:, EMB:] = lax.bitcast_convert_type(right, jnp.float32)


def _tc_pack(tt):
    return pl.pallas_call(
        _pack_body,
        grid=(NSTEP,),
        in_specs=[pl.BlockSpec((EMB, SLAB), lambda i: (0, i))],
        out_specs=pl.BlockSpec((Q, 2 * EMB), lambda i: (i, 0)),
        out_shape=jax.ShapeDtypeStruct((PACK_ROWS, 2 * EMB), jnp.float32),
        compiler_params=pltpu.CompilerParams(
            dimension_semantics=("parallel",)),
    )(tt)


def _gather_body(p_hbm, jdx_hbm, x_hbm, jdx_v, rows_v, sem):
    wid = lax.axis_index("s") * NC + lax.axis_index("c")
    base = wid * B_PER_W
    pltpu.sync_copy(jdx_hbm.at[wid], jdx_v)
    copies = []
    for m in range(NCHUNK):
        copies.append(pltpu.async_copy(
            p_hbm.at[jdx_v.at[m]], rows_v.at[pl.ds(m * CHUNK, CHUNK)], sem))
    for c in copies:
        c.wait()
    pltpu.sync_copy(rows_v, x_hbm.at[pl.ds(base, B_PER_W)])


def _sc_gather(p, jdx3):
    mesh = plsc.VectorSubcoreMesh(core_axis_name="c", subcore_axis_name="s")
    scratch = [
        pltpu.VMEM((NCHUNK, CHUNK), jnp.int32),
        pltpu.VMEM((B_PER_W, 2 * EMB), jnp.float32),
        pltpu.SemaphoreType.DMA,
    ]
    k = pl.kernel(_gather_body,
                  out_type=jax.ShapeDtypeStruct((BATCH, 2 * EMB), jnp.float32),
                  mesh=mesh, scratch_types=scratch,
                  compiler_params=pltpu.CompilerParams(
                      use_tc_tiling_on_sc=True))
    return k(p, jdx3)


BLK = 2048


def _unpack(x_ref, half_ref, lo_ref):
    xi = lax.bitcast_convert_type(x_ref[...], jnp.uint32)
    half = half_ref[...] > 0.5
    lo = lo_ref[...] > 0.5
    w = jnp.where(half, xi[:, EMB:], xi[:, :EMB])
    bits = jnp.where(lo, lax.shift_left(w, jnp.uint32(16)),
                     w & jnp.uint32(0xFFFF0000))
    return lax.bitcast_convert_type(bits, jnp.float32)


def _mlp_body(xu_ref, xi_ref, uh_ref, ul_ref, ih_ref, il_ref, w1u_ref,
              w1i_ref, b1_ref, w2_ref, b2_ref, w3_ref, b3_ref, out_ref):
    ue = _unpack(xu_ref, uh_ref, ul_ref)
    ie = _unpack(xi_ref, ih_ref, il_ref)
    h1 = jnp.dot(ue, w1u_ref[...], preferred_element_type=jnp.float32)
    h1 += jnp.dot(ie, w1i_ref[...], preferred_element_type=jnp.float32)
    h1 = jnp.maximum(h1 + b1_ref[...], 0.0)
    h2 = jnp.dot(h1, w2_ref[...], preferred_element_type=jnp.float32)
    h2 = jnp.maximum(h2 + b2_ref[...], 0.0)
    out = jnp.dot(h2, w3_ref[...], preferred_element_type=jnp.float32)
    out_ref[...] = out[:, 0] + b3_ref[0]


def _tc_mlp(xu, xi, uh, ul, ih, il, W1u, W1i, b1r, W2, b2r, W3, b3):
    grid = (BATCH // BLK,)
    const = lambda shape: pl.BlockSpec(shape, lambda i: tuple(0 for _ in shape))
    col = pl.BlockSpec((BLK, 1), lambda i: (i, 0))
    return pl.pallas_call(
        _mlp_body,
        grid=grid,
        in_specs=[
            pl.BlockSpec((BLK, 2 * EMB), lambda i: (i, 0)),
            pl.BlockSpec((BLK, 2 * EMB), lambda i: (i, 0)),
            col, col, col, col,
            const((EMB, H1)),
            const((EMB, H1)),
            const((1, H1)),
            const((H1, H2)),
            const((1, H2)),
            const((H2, 1)),
            const((1,)),
        ],
        out_specs=pl.BlockSpec((BLK,), lambda i: (i,)),
        out_shape=jax.ShapeDtypeStruct((BATCH,), jnp.float32),
    )(xu, xi, uh, ul, ih, il, W1u, W1i, b1r, W2, b2r, W3, b3)


QB = Q.bit_length() - 1


def _row_ids(r):
    return ((r >> (QB + 2)) << QB) | (r & (Q - 1))


def _colmask(bits):
    return bits.astype(jnp.float32).reshape(BATCH, 1)


@jax.jit
def kernel(user_table, item_table, W1, b1, W2, b2, W3, b3, user_input,
           item_input):
    ujdx = _row_ids(user_input)
    ijdx = _row_ids(item_input)
    pu = _tc_pack(user_table.T)
    xu = _sc_gather(pu, ujdx.reshape(NW, NCHUNK, CHUNK))
    pi = _tc_pack(item_table.T)
    xi = _sc_gather(pi, ijdx.reshape(NW, NCHUNK, CHUNK))
    uh = _colmask((user_input >> (QB + 1)) & 1)
    ul = _colmask((user_input >> QB) & 1)
    ih = _colmask((item_input >> (QB + 1)) & 1)
    il = _colmask((item_input >> QB) & 1)
    return _tc_mlp(xu, xi, uh, ul, ih, il, W1[:EMB], W1[EMB:],
                   b1.reshape(1, H1), W2, b2.reshape(1, H2), W3, b3)
